# baseline (speedup 1.0000x reference)
import jax
import jax.numpy as jnp
from jax.experimental import pallas as pl
from jax.experimental.pallas import tpu as pltpu

_B = 16


def _eye(n, dtype):
    return (jax.lax.broadcasted_iota(jnp.int32, (n, n), 0)
            == jax.lax.broadcasted_iota(jnp.int32, (n, n), 1)).astype(dtype)


def _body(ids_ref, x_ref, att_ref, bias_ref, out_ref, m_ref, s_ref):
    i = pl.program_id(0)
    nb = pl.num_programs(0)

    @pl.when(i == 0)
    def _init():
        m_ref[...] = jnp.full_like(m_ref, -jnp.inf)
        s_ref[...] = jnp.zeros_like(s_ref)
        out_ref[...] = jnp.zeros_like(out_ref)

    x = x_ref[...]
    l = jax.lax.dot_general(x, att_ref[...], (((1,), (0,)), ((), ())),
                            preferred_element_type=jnp.float32)
    l = l + bias_ref[0, 0]
    ids = ids_ref[...]
    oh = ids == jax.lax.broadcasted_iota(jnp.int32, (1, _B), 1)

    m_old = m_ref[...]
    bm = jnp.max(jnp.where(oh, l, -jnp.inf), axis=0, keepdims=True)
    m_new = jnp.maximum(m_old, bm)
    scale = jnp.where(m_old == -jnp.inf, 0.0, jnp.exp(m_old - m_new))
    p = jnp.exp(jnp.where(oh, l - m_new, -jnp.inf))

    s_ref[...] = s_ref[...] * scale + jnp.sum(p, axis=0, keepdims=True)
    m_ref[...] = m_new

    eye = _eye(_B, jnp.float32)
    scale_col = jax.lax.dot_general(eye, scale, (((1,), (1,)), ((), ())),
                                    preferred_element_type=jnp.float32)
    pTx = jax.lax.dot_general(p, x, (((0,), (0,)), ((), ())),
                              preferred_element_type=jnp.float32)
    out_ref[...] = out_ref[...] * scale_col + pTx

    @pl.when(i == nb - 1)
    def _fin():
        s_col = jax.lax.dot_general(eye, s_ref[...], (((1,), (1,)), ((), ())),
                                    preferred_element_type=jnp.float32)
        inv = jnp.where(s_col > 0.0, 1.0 / s_col, 0.0)
        out_ref[...] = out_ref[...] * inv


def _run(ids, flat, att2, bias2, blk):
    n, d = flat.shape
    return pl.pallas_call(
        _body,
        grid=(n // blk,),
        in_specs=[
            pl.BlockSpec((blk, 1), lambda i: (i, 0)),
            pl.BlockSpec((blk, d), lambda i: (i, 0)),
            pl.BlockSpec((d, 1), lambda i: (0, 0)),
            pl.BlockSpec((1, 1), lambda i: (0, 0)),
        ],
        out_specs=pl.BlockSpec((_B, d), lambda i: (0, 0)),
        out_shape=jax.ShapeDtypeStruct((_B, d), jnp.float32),
        scratch_shapes=[
            pltpu.VMEM((1, _B), jnp.float32),
            pltpu.VMEM((1, _B), jnp.float32),
        ],
    )(ids, flat, att2, bias2)


@jax.jit
def kernel(flat, segment_ids, att, bias, temperature):
    n, _ = flat.shape
    att2 = att * temperature[0]
    bias2 = (bias[0] * temperature[0]).reshape(1, 1)
    ids = segment_ids.astype(jnp.int32).reshape(n, 1)
    return _run(ids, flat, att2, bias2, 4096)


# speedup vs baseline: 1.0114x; 1.0114x over previous
import jax
import jax.numpy as jnp
from jax.experimental import pallas as pl
from jax.experimental.pallas import tpu as pltpu

_B = 16


def _eye(n, dtype):
    return (jax.lax.broadcasted_iota(jnp.int32, (n, n), 0)
            == jax.lax.broadcasted_iota(jnp.int32, (n, n), 1)).astype(dtype)


def _body(ids_ref, x_ref, att_ref, bias_ref, out_ref, m_ref, s_ref):
    i = pl.program_id(0)
    nb = pl.num_programs(0)

    @pl.when(i == 0)
    def _init():
        m_ref[...] = jnp.full_like(m_ref, -jnp.inf)
        s_ref[...] = jnp.zeros_like(s_ref)
        out_ref[...] = jnp.zeros_like(out_ref)

    x = x_ref[...]
    l = jax.lax.dot_general(x, att_ref[...], (((1,), (0,)), ((), ())),
                            preferred_element_type=jnp.float32)
    l = l + bias_ref[0, 0]
    ids = ids_ref[...]
    oh = ids == jax.lax.broadcasted_iota(jnp.int32, (1, _B), 1)

    m_old = m_ref[...]
    bm = jnp.max(jnp.where(oh, l, -jnp.inf), axis=0, keepdims=True)
    m_new = jnp.maximum(m_old, bm)
    scale = jnp.where(m_old == -jnp.inf, 0.0, jnp.exp(m_old - m_new))
    p = jnp.exp(jnp.where(oh, l - m_new, -jnp.inf))

    s_ref[...] = s_ref[...] * scale + jnp.sum(p, axis=0, keepdims=True)
    m_ref[...] = m_new

    eye = _eye(_B, jnp.float32)
    scale_col = jax.lax.dot_general(eye, scale, (((1,), (1,)), ((), ())),
                                    preferred_element_type=jnp.float32)
    pTx = jax.lax.dot_general(p, x, (((0,), (0,)), ((), ())),
                              preferred_element_type=jnp.float32)
    out_ref[...] = out_ref[...] * scale_col + pTx

    @pl.when(i == nb - 1)
    def _fin():
        s_col = jax.lax.dot_general(eye, s_ref[...], (((1,), (1,)), ((), ())),
                                    preferred_element_type=jnp.float32)
        inv = jnp.where(s_col > 0.0, 1.0 / s_col, 0.0)
        out_ref[...] = out_ref[...] * inv


def _run(ids, flat, att2, bias2, blk):
    n, d = flat.shape
    return pl.pallas_call(
        _body,
        grid=(n // blk,),
        in_specs=[
            pl.BlockSpec((blk, 1), lambda i: (i, 0)),
            pl.BlockSpec((blk, d), lambda i: (i, 0)),
            pl.BlockSpec((d, 1), lambda i: (0, 0)),
            pl.BlockSpec((1, 1), lambda i: (0, 0)),
        ],
        out_specs=pl.BlockSpec((_B, d), lambda i: (0, 0)),
        out_shape=jax.ShapeDtypeStruct((_B, d), jnp.float32),
        scratch_shapes=[
            pltpu.VMEM((1, _B), jnp.float32),
            pltpu.VMEM((1, _B), jnp.float32),
        ],
    )(ids, flat, att2, bias2)


@jax.jit
def kernel(flat, segment_ids, att, bias, temperature):
    n, _ = flat.shape
    att2 = att * temperature[0]
    bias2 = (bias[0] * temperature[0]).reshape(1, 1)
    ids = segment_ids.astype(jnp.int32).reshape(n, 1)
    return _run(ids, flat, att2, bias2, 2048)


# speedup vs baseline: 1.0138x; 1.0023x over previous
import jax
import jax.numpy as jnp
from jax.experimental import pallas as pl
from jax.experimental.pallas import tpu as pltpu

_B = 16


def _eye(n, dtype):
    return (jax.lax.broadcasted_iota(jnp.int32, (n, n), 0)
            == jax.lax.broadcasted_iota(jnp.int32, (n, n), 1)).astype(dtype)


def _body(ids_ref, x_ref, att_ref, bias_ref, out_ref, m_ref, s_ref):
    i = pl.program_id(0)
    nb = pl.num_programs(0)

    @pl.when(i == 0)
    def _init():
        m_ref[...] = jnp.full_like(m_ref, -jnp.inf)
        s_ref[...] = jnp.zeros_like(s_ref)
        out_ref[...] = jnp.zeros_like(out_ref)

    x = x_ref[...]
    xh = x.astype(jnp.bfloat16)
    l = jax.lax.dot_general(xh, att_ref[...].astype(jnp.bfloat16),
                            (((1,), (0,)), ((), ())),
                            preferred_element_type=jnp.float32)
    l = l + bias_ref[0, 0]
    ids = ids_ref[...]
    oh = ids == jax.lax.broadcasted_iota(jnp.int32, (1, _B), 1)

    m_old = m_ref[...]
    bm = jnp.max(jnp.where(oh, l, -jnp.inf), axis=0, keepdims=True)
    m_new = jnp.maximum(m_old, bm)
    scale = jnp.where(m_old == -jnp.inf, 0.0, jnp.exp(m_old - m_new))
    p = jnp.exp(jnp.where(oh, l - m_new, -jnp.inf))

    s_ref[...] = s_ref[...] * scale + jnp.sum(p, axis=0, keepdims=True)
    m_ref[...] = m_new

    eye = _eye(_B, jnp.float32)
    scale_col = jax.lax.dot_general(eye, scale, (((1,), (1,)), ((), ())),
                                    preferred_element_type=jnp.float32)
    pTx = jax.lax.dot_general(p.astype(jnp.bfloat16), xh,
                              (((0,), (0,)), ((), ())),
                              preferred_element_type=jnp.float32)
    out_ref[...] = out_ref[...] * scale_col + pTx

    @pl.when(i == nb - 1)
    def _fin():
        s_col = jax.lax.dot_general(eye, s_ref[...], (((1,), (1,)), ((), ())),
                                    preferred_element_type=jnp.float32)
        inv = jnp.where(s_col > 0.0, 1.0 / s_col, 0.0)
        out_ref[...] = out_ref[...] * inv


def _run(ids, flat, att2, bias2, blk):
    n, d = flat.shape
    return pl.pallas_call(
        _body,
        grid=(n // blk,),
        in_specs=[
            pl.BlockSpec((blk, 1), lambda i: (i, 0)),
            pl.BlockSpec((blk, d), lambda i: (i, 0)),
            pl.BlockSpec((d, 1), lambda i: (0, 0)),
            pl.BlockSpec((1, 1), lambda i: (0, 0)),
        ],
        out_specs=pl.BlockSpec((_B, d), lambda i: (0, 0)),
        out_shape=jax.ShapeDtypeStruct((_B, d), jnp.float32),
        scratch_shapes=[
            pltpu.VMEM((1, _B), jnp.float32),
            pltpu.VMEM((1, _B), jnp.float32),
        ],
    )(ids, flat, att2, bias2)


@jax.jit
def kernel(flat, segment_ids, att, bias, temperature):
    n, _ = flat.shape
    att2 = att * temperature[0]
    bias2 = (bias[0] * temperature[0]).reshape(1, 1)
    ids = segment_ids.astype(jnp.int32).reshape(n, 1)
    return _run(ids, flat, att2, bias2, 2048)


# speedup vs baseline: 1.0657x; 1.0512x over previous
import jax
import jax.numpy as jnp
from jax.experimental import pallas as pl
from jax.experimental.pallas import tpu as pltpu

_B = 16


def _eye(n, dtype):
    return (jax.lax.broadcasted_iota(jnp.int32, (n, n), 0)
            == jax.lax.broadcasted_iota(jnp.int32, (n, n), 1)).astype(dtype)


def _body(ids_ref, x1_ref, x2_ref, att_ref, bias_ref, out1_ref, out2_ref,
          m_ref, s_ref):
    i = pl.program_id(0)
    nb = pl.num_programs(0)

    @pl.when(i == 0)
    def _init():
        m_ref[...] = jnp.full_like(m_ref, -jnp.inf)
        s_ref[...] = jnp.zeros_like(s_ref)
        out1_ref[...] = jnp.zeros_like(out1_ref)
        out2_ref[...] = jnp.zeros_like(out2_ref)

    x1 = x1_ref[...].astype(jnp.bfloat16)
    x2 = x2_ref[...].astype(jnp.bfloat16)
    att = att_ref[...].astype(jnp.bfloat16)
    hd = x1.shape[1]
    dn = (((1,), (0,)), ((), ()))
    l = (jax.lax.dot_general(x1, att[:hd], dn, preferred_element_type=jnp.float32)
         + jax.lax.dot_general(x2, att[hd:], dn, preferred_element_type=jnp.float32))
    l = l + bias_ref[0, 0]
    ids = ids_ref[...]
    oh = ids == jax.lax.broadcasted_iota(jnp.int32, (1, _B), 1)

    m_old = m_ref[...]
    bm = jnp.max(jnp.where(oh, l, -jnp.inf), axis=0, keepdims=True)
    m_new = jnp.maximum(m_old, bm)
    scale = jnp.where(m_old == -jnp.inf, 0.0, jnp.exp(m_old - m_new))
    p = jnp.exp(jnp.where(oh, l - m_new, -jnp.inf))

    s_ref[...] = s_ref[...] * scale + jnp.sum(p, axis=0, keepdims=True)
    m_ref[...] = m_new

    eye = _eye(_B, jnp.float32)
    tdn = (((1,), (1,)), ((), ()))
    scale_col = jax.lax.dot_general(eye, scale, tdn,
                                    preferred_element_type=jnp.float32)
    ph = p.astype(jnp.bfloat16)
    cdn = (((0,), (0,)), ((), ()))
    out1_ref[...] = out1_ref[...] * scale_col + jax.lax.dot_general(
        ph, x1, cdn, preferred_element_type=jnp.float32)
    out2_ref[...] = out2_ref[...] * scale_col + jax.lax.dot_general(
        ph, x2, cdn, preferred_element_type=jnp.float32)

    @pl.when(i == nb - 1)
    def _fin():
        s_col = jax.lax.dot_general(eye, s_ref[...], tdn,
                                    preferred_element_type=jnp.float32)
        inv = jnp.where(s_col > 0.0, 1.0 / s_col, 0.0)
        out1_ref[...] = out1_ref[...] * inv
        out2_ref[...] = out2_ref[...] * inv


def _run(ids, flat, att2, bias2, blk):
    n, d = flat.shape
    hd = d // 2
    o1, o2 = pl.pallas_call(
        _body,
        grid=(n // blk,),
        in_specs=[
            pl.BlockSpec((blk, 1), lambda i: (i, 0)),
            pl.BlockSpec((blk, hd), lambda i: (i, 0)),
            pl.BlockSpec((blk, hd), lambda i: (i, 1)),
            pl.BlockSpec((d, 1), lambda i: (0, 0)),
            pl.BlockSpec((1, 1), lambda i: (0, 0)),
        ],
        out_specs=[
            pl.BlockSpec((_B, hd), lambda i: (0, 0)),
            pl.BlockSpec((_B, hd), lambda i: (0, 0)),
        ],
        out_shape=[
            jax.ShapeDtypeStruct((_B, hd), jnp.float32),
            jax.ShapeDtypeStruct((_B, hd), jnp.float32),
        ],
        scratch_shapes=[
            pltpu.VMEM((1, _B), jnp.float32),
            pltpu.VMEM((1, _B), jnp.float32),
        ],
    )(ids, flat, flat, att2, bias2)
    return jnp.concatenate([o1, o2], axis=1)


@jax.jit
def kernel(flat, segment_ids, att, bias, temperature):
    n, _ = flat.shape
    att2 = att * temperature[0]
    bias2 = (bias[0] * temperature[0]).reshape(1, 1)
    ids = segment_ids.astype(jnp.int32).reshape(n, 1)
    return _run(ids, flat, att2, bias2, 2048)
